# baseline (speedup 1.0000x reference)
import numpy as np
import jax
import jax.numpy as jnp
from jax.experimental import pallas as pl
from jax.experimental.pallas import tpu as pltpu

_F32 = jnp.float32


def _gather_body(refs_smem, train_ref, mem_ref, oimg_ref, omem_ref):
    oimg_ref[...] = train_ref[...]
    omem_ref[...] = mem_ref[...]


def _featurize_body(x_ref, T1_ref, b1_ref, T2_ref, b2_ref, o_ref):
    B = x_ref.shape[0]
    x = x_ref[...]
    T1 = T1_ref[...]
    T2 = T2_ref[...]
    acc = None
    for c in range(3):
        for d in range(5):
            A = x[:, c, d:d + 28, :].reshape(B * 28, 32)
            t = jnp.dot(A, T1[c * 5 + d], preferred_element_type=_F32)
            acc = t if acc is None else acc + t
    acc = jnp.maximum(acc + b1_ref[...], 0.0)
    pj = jnp.maximum(acc[:, :84], acc[:, 84:])
    pj = jnp.max(pj.reshape(B * 14, 2, 84), axis=1)
    p1 = pj.reshape(B, 14, 84)
    acc2 = None
    for c in range(6):
        for d in range(5):
            A2 = p1[:, d:d + 10, c * 14:(c + 1) * 14].reshape(B * 10, 14)
            t = jnp.dot(A2, T2[c * 5 + d], preferred_element_type=_F32)
            acc2 = t if acc2 is None else acc2 + t
    acc2 = jnp.maximum(acc2 + b2_ref[...], 0.0)
    qj = jnp.maximum(acc2[:, :80], acc2[:, 80:])
    qj = jnp.max(qj.reshape(B * 5, 2, 80), axis=1)
    o_ref[...] = qj.reshape(B, 5, 80)


def _combine_body(fi_ref, fr_ref, gm_ref, Wfs_ref, bf_ref, Wgs_ref, bg_ref,
                  Wmt_ref, bm_ref, o_ref):
    B = fi_ref.shape[0]
    S = fr_ref.shape[0] // B
    H = bf_ref.shape[1]
    C = bm_ref.shape[1]
    fi = fi_ref[...]
    fr = fr_ref[...]
    Wfs = Wfs_ref[...]
    Wgs = Wgs_ref[...]
    xs = rs = gs = None
    for i in range(5):
        a = jnp.dot(fi[:, i, :], Wfs[i], preferred_element_type=_F32)
        r = jnp.dot(fr[:, i, :], Wfs[i], preferred_element_type=_F32)
        g = jnp.dot(fi[:, i, :], Wgs[i], preferred_element_type=_F32)
        xs = a if xs is None else xs + a
        rs = r if rs is None else rs + r
        gs = g if gs is None else gs + g
    x = xs + bf_ref[...]
    rx = (rs + bf_ref[...]).reshape(B, S, H)
    g = gs + bg_ref[...]
    diff = x[:, None, :] - rx
    ml = jnp.dot(gm_ref[...].reshape(B * S, C), Wmt_ref[...],
                 preferred_element_type=_F32) + bm_ref[...]
    lg = ml.reshape(B, S, C)
    for h in range(H):
        gh = g[:, h * C:(h + 1) * C][:, None, :]
        lg = lg + diff[:, :, h:h + 1] * gh
    nrm = jnp.sqrt(jnp.sum(diff * diff, axis=-1))
    mx = jnp.max(-nrm, axis=-1, keepdims=True)
    e = jnp.exp(-nrm - mx)
    dist = e / jnp.sum(e, axis=-1, keepdims=True)
    o_ref[...] = jnp.sum(lg * dist[:, :, None], axis=1)


def kernel(img, train, mem, w1, b1, w2, b2, Wf, bf, Wg, bg, Wm, bm, idx, refs):
    n, S = refs.shape
    H = bf.shape[0]
    C = bm.shape[0]
    nr = n * S

    s1 = np.zeros((5, 32, 28), np.float32)
    s2 = np.zeros((5, 14, 10), np.float32)
    for dj in range(5):
        s1[dj, dj + np.arange(28), np.arange(28)] = 1.0
        s2[dj, dj + np.arange(10), np.arange(10)] = 1.0
    T1 = jnp.einsum('Dwj,ocdD->cdwoj', jnp.asarray(s1), w1).reshape(15, 32, 168)
    T2 = jnp.einsum('Dwj,ocdD->cdwoj', jnp.asarray(s2), w2).reshape(30, 14, 160)
    O1, J1 = np.meshgrid(np.arange(6), np.arange(28), indexing='ij')
    p1m = np.empty(168, np.int32)
    p1m[((J1 % 2) * 84 + O1 * 14 + J1 // 2).ravel()] = np.arange(168)
    O2, J2 = np.meshgrid(np.arange(16), np.arange(10), indexing='ij')
    p2m = np.empty(160, np.int32)
    p2m[((J2 % 2) * 80 + O2 * 5 + J2 // 2).ravel()] = np.arange(160)
    T1 = T1[:, :, p1m]
    T2 = T2[:, :, p2m]
    b1r = jnp.tile(jnp.repeat(b1, 14), 2).reshape(1, 168)
    b2r = jnp.tile(jnp.repeat(b2, 5), 2).reshape(1, 160)
    perm = np.arange(400).reshape(16, 5, 5).transpose(1, 0, 2).reshape(400)
    Wfs = Wf[:, perm].reshape(H, 5, 80).transpose(1, 2, 0)
    Wgs = Wg[:, perm].reshape(H * C, 5, 80).transpose(1, 2, 0)
    bf2 = bf.reshape(1, H)
    bg2 = bg.reshape(1, H * C)
    Wmt = Wm.T
    bm2 = bm.reshape(1, C)
    refs_flat = refs.reshape(-1).astype(jnp.int32)
    mem3 = mem.reshape(mem.shape[0], 1, C)

    gi, gm = pl.pallas_call(
        _gather_body,
        grid_spec=pltpu.PrefetchScalarGridSpec(
            num_scalar_prefetch=1,
            grid=(nr,),
            in_specs=[
                pl.BlockSpec((1, 3, 32, 32), lambda i, r: (r[i], 0, 0, 0)),
                pl.BlockSpec((1, 1, C), lambda i, r: (r[i], 0, 0)),
            ],
            out_specs=[
                pl.BlockSpec((1, 3, 32, 32), lambda i, r: (i, 0, 0, 0)),
                pl.BlockSpec((1, 1, C), lambda i, r: (i, 0, 0)),
            ],
        ),
        out_shape=[
            jax.ShapeDtypeStruct((nr, 3, 32, 32), _F32),
            jax.ShapeDtypeStruct((nr, 1, C), _F32),
        ],
        compiler_params=pltpu.CompilerParams(
            dimension_semantics=("arbitrary",)),
        name="mad_gather",
    )(refs_flat, train, mem3)

    def featurize(x, B):
        nb = x.shape[0] // B
        return pl.pallas_call(
            _featurize_body,
            grid=(nb,),
            in_specs=[
                pl.BlockSpec((B, 3, 32, 32), lambda b: (b, 0, 0, 0)),
                pl.BlockSpec((15, 32, 168), lambda b: (0, 0, 0)),
                pl.BlockSpec((1, 168), lambda b: (0, 0)),
                pl.BlockSpec((30, 14, 160), lambda b: (0, 0, 0)),
                pl.BlockSpec((1, 160), lambda b: (0, 0)),
            ],
            out_specs=pl.BlockSpec((B, 5, 80), lambda b: (b, 0, 0)),
            out_shape=jax.ShapeDtypeStruct((x.shape[0], 5, 80), _F32),
            compiler_params=pltpu.CompilerParams(
                dimension_semantics=("parallel",)),
            name="mad_featurize",
        )(x, T1, b1r, T2, b2r)

    fi = featurize(img, 128)
    fr = featurize(gi, 128)

    BN = 128
    out = pl.pallas_call(
        _combine_body,
        grid=(n // BN,),
        in_specs=[
            pl.BlockSpec((BN, 5, 80), lambda b: (b, 0, 0)),
            pl.BlockSpec((BN * S, 5, 80), lambda b: (b, 0, 0)),
            pl.BlockSpec((BN * S, 1, C), lambda b: (b, 0, 0)),
            pl.BlockSpec((5, 80, H), lambda b: (0, 0, 0)),
            pl.BlockSpec((1, H), lambda b: (0, 0)),
            pl.BlockSpec((5, 80, H * C), lambda b: (0, 0, 0)),
            pl.BlockSpec((1, H * C), lambda b: (0, 0)),
            pl.BlockSpec((C, C), lambda b: (0, 0)),
            pl.BlockSpec((1, C), lambda b: (0, 0)),
        ],
        out_specs=pl.BlockSpec((BN, C), lambda b: (b, 0)),
        out_shape=jax.ShapeDtypeStruct((n, C), _F32),
        compiler_params=pltpu.CompilerParams(
            dimension_semantics=("parallel",)),
        name="mad_combine",
    )(fi, fr, gm, Wfs, bf2, Wgs, bg2, Wmt, bm2)
    return out


# speedup vs baseline: 1.8236x; 1.8236x over previous
import numpy as np
import jax
import jax.numpy as jnp
from jax.experimental import pallas as pl
from jax.experimental.pallas import tpu as pltpu

_F32 = jnp.float32


_G = 16


def _gather_body(refs_smem, *refs_args):
    timg = refs_args[:_G]
    tmem = refs_args[_G:2 * _G]
    oimg_ref = refs_args[2 * _G]
    omem_ref = refs_args[2 * _G + 1]
    for g in range(_G):
        oimg_ref[g] = timg[g][0]
        omem_ref[g] = tmem[g][0]


def _featurize_body(x_ref, T1_ref, b1_ref, T2_ref, b2_ref, o_ref):
    B = x_ref.shape[0]
    x = x_ref[...]
    T1 = T1_ref[...]
    T2 = T2_ref[...]
    acc = None
    for c in range(3):
        for d in range(5):
            A = x[:, c, d:d + 28, :].reshape(B * 28, 32)
            t = jnp.dot(A, T1[c * 5 + d], preferred_element_type=_F32)
            acc = t if acc is None else acc + t
    acc = jnp.maximum(acc + b1_ref[...], 0.0)
    pj = jnp.maximum(acc[:, :84], acc[:, 84:])
    pj = jnp.max(pj.reshape(B * 14, 2, 84), axis=1)
    p1 = pj.reshape(B, 14, 84)
    acc2 = None
    for c in range(6):
        for d in range(5):
            A2 = p1[:, d:d + 10, c * 14:(c + 1) * 14].reshape(B * 10, 14)
            t = jnp.dot(A2, T2[c * 5 + d], preferred_element_type=_F32)
            acc2 = t if acc2 is None else acc2 + t
    acc2 = jnp.maximum(acc2 + b2_ref[...], 0.0)
    qj = jnp.maximum(acc2[:, :80], acc2[:, 80:])
    qj = jnp.max(qj.reshape(B * 5, 2, 80), axis=1)
    o_ref[...] = qj.reshape(B, 5, 80)


def _combine_body(fi_ref, fr_ref, gm_ref, Wfs_ref, bf_ref, Wgs_ref, bg_ref,
                  Wmt_ref, bm_ref, o_ref):
    B = fi_ref.shape[0]
    S = fr_ref.shape[0] // B
    H = bf_ref.shape[1]
    C = bm_ref.shape[1]
    fi = fi_ref[...]
    fr = fr_ref[...]
    Wfs = Wfs_ref[...]
    Wgs = Wgs_ref[...]
    xs = rs = gs = None
    for i in range(5):
        a = jnp.dot(fi[:, i, :], Wfs[i], preferred_element_type=_F32)
        r = jnp.dot(fr[:, i, :], Wfs[i], preferred_element_type=_F32)
        g = jnp.dot(fi[:, i, :], Wgs[i], preferred_element_type=_F32)
        xs = a if xs is None else xs + a
        rs = r if rs is None else rs + r
        gs = g if gs is None else gs + g
    x = xs + bf_ref[...]
    rx = (rs + bf_ref[...]).reshape(B, S, H)
    g = gs + bg_ref[...]
    diff = x[:, None, :] - rx
    ml = jnp.dot(gm_ref[...].reshape(B * S, C), Wmt_ref[...],
                 preferred_element_type=_F32) + bm_ref[...]
    lg = ml.reshape(B, S, C)
    for h in range(H):
        gh = g[:, h * C:(h + 1) * C][:, None, :]
        lg = lg + diff[:, :, h:h + 1] * gh
    nrm = jnp.sqrt(jnp.sum(diff * diff, axis=-1))
    mx = jnp.max(-nrm, axis=-1, keepdims=True)
    e = jnp.exp(-nrm - mx)
    dist = e / jnp.sum(e, axis=-1, keepdims=True)
    o_ref[...] = jnp.sum(lg * dist[:, :, None], axis=1)


def kernel(img, train, mem, w1, b1, w2, b2, Wf, bf, Wg, bg, Wm, bm, idx, refs):
    n, S = refs.shape
    H = bf.shape[0]
    C = bm.shape[0]
    nr = n * S

    s1 = np.zeros((5, 32, 28), np.float32)
    s2 = np.zeros((5, 14, 10), np.float32)
    for dj in range(5):
        s1[dj, dj + np.arange(28), np.arange(28)] = 1.0
        s2[dj, dj + np.arange(10), np.arange(10)] = 1.0
    T1 = jnp.einsum('Dwj,ocdD->cdwoj', jnp.asarray(s1), w1).reshape(15, 32, 168)
    T2 = jnp.einsum('Dwj,ocdD->cdwoj', jnp.asarray(s2), w2).reshape(30, 14, 160)
    O1, J1 = np.meshgrid(np.arange(6), np.arange(28), indexing='ij')
    p1m = np.empty(168, np.int32)
    p1m[((J1 % 2) * 84 + O1 * 14 + J1 // 2).ravel()] = np.arange(168)
    O2, J2 = np.meshgrid(np.arange(16), np.arange(10), indexing='ij')
    p2m = np.empty(160, np.int32)
    p2m[((J2 % 2) * 80 + O2 * 5 + J2 // 2).ravel()] = np.arange(160)
    T1 = T1[:, :, p1m]
    T2 = T2[:, :, p2m]
    b1r = jnp.tile(jnp.repeat(b1, 14), 2).reshape(1, 168)
    b2r = jnp.tile(jnp.repeat(b2, 5), 2).reshape(1, 160)
    perm = np.arange(400).reshape(16, 5, 5).transpose(1, 0, 2).reshape(400)
    Wfs = Wf[:, perm].reshape(H, 5, 80).transpose(1, 2, 0)
    Wgs = Wg[:, perm].reshape(H * C, 5, 80).transpose(1, 2, 0)
    bf2 = bf.reshape(1, H)
    bg2 = bg.reshape(1, H * C)
    Wmt = Wm.T
    bm2 = bm.reshape(1, C)
    refs_flat = refs.reshape(-1).astype(jnp.int32)
    mem3 = mem.reshape(mem.shape[0], 1, C)

    img_specs = [
        pl.BlockSpec((1, 3, 32, 32), lambda i, r, g=g: (r[i * _G + g], 0, 0, 0))
        for g in range(_G)
    ]
    mem_specs = [
        pl.BlockSpec((1, 1, C), lambda i, r, g=g: (r[i * _G + g], 0, 0))
        for g in range(_G)
    ]
    gi, gm = pl.pallas_call(
        _gather_body,
        grid_spec=pltpu.PrefetchScalarGridSpec(
            num_scalar_prefetch=1,
            grid=(nr // _G,),
            in_specs=img_specs + mem_specs,
            out_specs=[
                pl.BlockSpec((_G, 3, 32, 32), lambda i, r: (i, 0, 0, 0)),
                pl.BlockSpec((_G, 1, C), lambda i, r: (i, 0, 0)),
            ],
        ),
        out_shape=[
            jax.ShapeDtypeStruct((nr, 3, 32, 32), _F32),
            jax.ShapeDtypeStruct((nr, 1, C), _F32),
        ],
        compiler_params=pltpu.CompilerParams(
            dimension_semantics=("arbitrary",)),
        name="mad_gather",
    )(refs_flat, *([train] * _G), *([mem3] * _G))

    def featurize(x, B):
        nb = x.shape[0] // B
        return pl.pallas_call(
            _featurize_body,
            grid=(nb,),
            in_specs=[
                pl.BlockSpec((B, 3, 32, 32), lambda b: (b, 0, 0, 0)),
                pl.BlockSpec((15, 32, 168), lambda b: (0, 0, 0)),
                pl.BlockSpec((1, 168), lambda b: (0, 0)),
                pl.BlockSpec((30, 14, 160), lambda b: (0, 0, 0)),
                pl.BlockSpec((1, 160), lambda b: (0, 0)),
            ],
            out_specs=pl.BlockSpec((B, 5, 80), lambda b: (b, 0, 0)),
            out_shape=jax.ShapeDtypeStruct((x.shape[0], 5, 80), _F32),
            compiler_params=pltpu.CompilerParams(
                dimension_semantics=("parallel",)),
            name="mad_featurize",
        )(x, T1, b1r, T2, b2r)

    fi = featurize(img, 128)
    fr = featurize(gi, 128)

    BN = 128
    out = pl.pallas_call(
        _combine_body,
        grid=(n // BN,),
        in_specs=[
            pl.BlockSpec((BN, 5, 80), lambda b: (b, 0, 0)),
            pl.BlockSpec((BN * S, 5, 80), lambda b: (b, 0, 0)),
            pl.BlockSpec((BN * S, 1, C), lambda b: (b, 0, 0)),
            pl.BlockSpec((5, 80, H), lambda b: (0, 0, 0)),
            pl.BlockSpec((1, H), lambda b: (0, 0)),
            pl.BlockSpec((5, 80, H * C), lambda b: (0, 0, 0)),
            pl.BlockSpec((1, H * C), lambda b: (0, 0)),
            pl.BlockSpec((C, C), lambda b: (0, 0)),
            pl.BlockSpec((1, C), lambda b: (0, 0)),
        ],
        out_specs=pl.BlockSpec((BN, C), lambda b: (b, 0)),
        out_shape=jax.ShapeDtypeStruct((n, C), _F32),
        compiler_params=pltpu.CompilerParams(
            dimension_semantics=("parallel",)),
        name="mad_combine",
    )(fi, fr, gm, Wfs, bf2, Wgs, bg2, Wmt, bm2)
    return out


# speedup vs baseline: 1.8865x; 1.0345x over previous
import numpy as np
import jax
import jax.numpy as jnp
from jax.experimental import pallas as pl
from jax.experimental.pallas import tpu as pltpu

_F32 = jnp.float32


_G = 32


def _gather_body(refs_smem, *refs_args):
    timg = refs_args[:_G]
    tmem = refs_args[_G:2 * _G]
    oimg_ref = refs_args[2 * _G]
    omem_ref = refs_args[2 * _G + 1]
    for g in range(_G):
        oimg_ref[g] = timg[g][0]
        omem_ref[g] = tmem[g][0]


def _featurize_body(x_ref, T1_ref, b1_ref, T2_ref, b2_ref, o_ref):
    B = x_ref.shape[0]
    x = x_ref[...]
    T1 = T1_ref[...]
    T2 = T2_ref[...]
    acc = None
    for c in range(3):
        for d in range(5):
            A = x[:, c, d:d + 28, :].reshape(B * 28, 32)
            t = jnp.dot(A, T1[c * 5 + d], preferred_element_type=_F32)
            acc = t if acc is None else acc + t
    acc = jnp.maximum(acc + b1_ref[...], 0.0)
    pj = jnp.maximum(acc[:, :84], acc[:, 84:])
    pj = jnp.max(pj.reshape(B * 14, 2, 84), axis=1)
    p1 = pj.reshape(B, 14, 84)
    acc2 = None
    for c in range(6):
        for d in range(5):
            A2 = p1[:, d:d + 10, c * 14:(c + 1) * 14].reshape(B * 10, 14)
            t = jnp.dot(A2, T2[c * 5 + d], preferred_element_type=_F32)
            acc2 = t if acc2 is None else acc2 + t
    acc2 = jnp.maximum(acc2 + b2_ref[...], 0.0)
    qj = jnp.maximum(acc2[:, :80], acc2[:, 80:])
    qj = jnp.max(qj.reshape(B * 5, 2, 80), axis=1)
    o_ref[...] = qj.reshape(B, 5, 80)


def _combine_body(fi_ref, fr_ref, gm_ref, Wfs_ref, bf_ref, Wgs_ref, bg_ref,
                  Wmt_ref, bm_ref, o_ref):
    B = fi_ref.shape[0]
    S = fr_ref.shape[0] // B
    H = bf_ref.shape[1]
    C = bm_ref.shape[1]
    fi = fi_ref[...]
    fr = fr_ref[...]
    Wfs = Wfs_ref[...]
    Wgs = Wgs_ref[...]
    xs = rs = gs = None
    for i in range(5):
        a = jnp.dot(fi[:, i, :], Wfs[i], preferred_element_type=_F32)
        r = jnp.dot(fr[:, i, :], Wfs[i], preferred_element_type=_F32)
        g = jnp.dot(fi[:, i, :], Wgs[i], preferred_element_type=_F32)
        xs = a if xs is None else xs + a
        rs = r if rs is None else rs + r
        gs = g if gs is None else gs + g
    x = xs + bf_ref[...]
    rx = (rs + bf_ref[...]).reshape(B, S, H)
    g = gs + bg_ref[...]
    diff = x[:, None, :] - rx
    ml = jnp.dot(gm_ref[...].reshape(B * S, C), Wmt_ref[...],
                 preferred_element_type=_F32) + bm_ref[...]
    lg = ml.reshape(B, S, C)
    for h in range(H):
        gh = g[:, h * C:(h + 1) * C][:, None, :]
        lg = lg + diff[:, :, h:h + 1] * gh
    nrm = jnp.sqrt(jnp.sum(diff * diff, axis=-1))
    mx = jnp.max(-nrm, axis=-1, keepdims=True)
    e = jnp.exp(-nrm - mx)
    dist = e / jnp.sum(e, axis=-1, keepdims=True)
    o_ref[...] = jnp.sum(lg * dist[:, :, None], axis=1)


def kernel(img, train, mem, w1, b1, w2, b2, Wf, bf, Wg, bg, Wm, bm, idx, refs):
    n, S = refs.shape
    H = bf.shape[0]
    C = bm.shape[0]
    nr = n * S

    s1 = np.zeros((5, 32, 28), np.float32)
    s2 = np.zeros((5, 14, 10), np.float32)
    for dj in range(5):
        s1[dj, dj + np.arange(28), np.arange(28)] = 1.0
        s2[dj, dj + np.arange(10), np.arange(10)] = 1.0
    T1 = jnp.einsum('Dwj,ocdD->cdwoj', jnp.asarray(s1), w1).reshape(15, 32, 168)
    T2 = jnp.einsum('Dwj,ocdD->cdwoj', jnp.asarray(s2), w2).reshape(30, 14, 160)
    O1, J1 = np.meshgrid(np.arange(6), np.arange(28), indexing='ij')
    p1m = np.empty(168, np.int32)
    p1m[((J1 % 2) * 84 + O1 * 14 + J1 // 2).ravel()] = np.arange(168)
    O2, J2 = np.meshgrid(np.arange(16), np.arange(10), indexing='ij')
    p2m = np.empty(160, np.int32)
    p2m[((J2 % 2) * 80 + O2 * 5 + J2 // 2).ravel()] = np.arange(160)
    T1 = T1[:, :, p1m]
    T2 = T2[:, :, p2m]
    b1r = jnp.tile(jnp.repeat(b1, 14), 2).reshape(1, 168)
    b2r = jnp.tile(jnp.repeat(b2, 5), 2).reshape(1, 160)
    perm = np.arange(400).reshape(16, 5, 5).transpose(1, 0, 2).reshape(400)
    Wfs = Wf[:, perm].reshape(H, 5, 80).transpose(1, 2, 0)
    Wgs = Wg[:, perm].reshape(H * C, 5, 80).transpose(1, 2, 0)
    bf2 = bf.reshape(1, H)
    bg2 = bg.reshape(1, H * C)
    Wmt = Wm.T
    bm2 = bm.reshape(1, C)
    refs_flat = refs.reshape(-1).astype(jnp.int32)
    mem3 = mem.reshape(mem.shape[0], 1, C)

    img_specs = [
        pl.BlockSpec((1, 3, 32, 32), lambda i, r, g=g: (r[i * _G + g], 0, 0, 0))
        for g in range(_G)
    ]
    mem_specs = [
        pl.BlockSpec((1, 1, C), lambda i, r, g=g: (r[i * _G + g], 0, 0))
        for g in range(_G)
    ]
    gi, gm = pl.pallas_call(
        _gather_body,
        grid_spec=pltpu.PrefetchScalarGridSpec(
            num_scalar_prefetch=1,
            grid=(nr // _G,),
            in_specs=img_specs + mem_specs,
            out_specs=[
                pl.BlockSpec((_G, 3, 32, 32), lambda i, r: (i, 0, 0, 0)),
                pl.BlockSpec((_G, 1, C), lambda i, r: (i, 0, 0)),
            ],
        ),
        out_shape=[
            jax.ShapeDtypeStruct((nr, 3, 32, 32), _F32),
            jax.ShapeDtypeStruct((nr, 1, C), _F32),
        ],
        compiler_params=pltpu.CompilerParams(
            dimension_semantics=("arbitrary",)),
        name="mad_gather",
    )(refs_flat, *([train] * _G), *([mem3] * _G))

    def featurize(x, B):
        nb = x.shape[0] // B
        return pl.pallas_call(
            _featurize_body,
            grid=(nb,),
            in_specs=[
                pl.BlockSpec((B, 3, 32, 32), lambda b: (b, 0, 0, 0)),
                pl.BlockSpec((15, 32, 168), lambda b: (0, 0, 0)),
                pl.BlockSpec((1, 168), lambda b: (0, 0)),
                pl.BlockSpec((30, 14, 160), lambda b: (0, 0, 0)),
                pl.BlockSpec((1, 160), lambda b: (0, 0)),
            ],
            out_specs=pl.BlockSpec((B, 5, 80), lambda b: (b, 0, 0)),
            out_shape=jax.ShapeDtypeStruct((x.shape[0], 5, 80), _F32),
            compiler_params=pltpu.CompilerParams(
                dimension_semantics=("parallel",)),
            name="mad_featurize",
        )(x, T1, b1r, T2, b2r)

    fi = featurize(img, 128)
    fr = featurize(gi, 128)

    BN = 128
    out = pl.pallas_call(
        _combine_body,
        grid=(n // BN,),
        in_specs=[
            pl.BlockSpec((BN, 5, 80), lambda b: (b, 0, 0)),
            pl.BlockSpec((BN * S, 5, 80), lambda b: (b, 0, 0)),
            pl.BlockSpec((BN * S, 1, C), lambda b: (b, 0, 0)),
            pl.BlockSpec((5, 80, H), lambda b: (0, 0, 0)),
            pl.BlockSpec((1, H), lambda b: (0, 0)),
            pl.BlockSpec((5, 80, H * C), lambda b: (0, 0, 0)),
            pl.BlockSpec((1, H * C), lambda b: (0, 0)),
            pl.BlockSpec((C, C), lambda b: (0, 0)),
            pl.BlockSpec((1, C), lambda b: (0, 0)),
        ],
        out_specs=pl.BlockSpec((BN, C), lambda b: (b, 0)),
        out_shape=jax.ShapeDtypeStruct((n, C), _F32),
        compiler_params=pltpu.CompilerParams(
            dimension_semantics=("parallel",)),
        name="mad_combine",
    )(fi, fr, gm, Wfs, bf2, Wgs, bg2, Wmt, bm2)
    return out


# speedup vs baseline: 1.8870x; 1.0003x over previous
import numpy as np
import jax
import jax.numpy as jnp
from jax.experimental import pallas as pl
from jax.experimental.pallas import tpu as pltpu

_F32 = jnp.float32


_G = 32


def _gather_body(refs_smem, *refs_args):
    timg = refs_args[:_G]
    tmem = refs_args[_G:2 * _G]
    oimg_ref = refs_args[2 * _G]
    omem_ref = refs_args[2 * _G + 1]
    for g in range(_G):
        oimg_ref[g] = timg[g][0]
        omem_ref[g] = tmem[g][0]


def _featurize_body(x_ref, T1_ref, b1_ref, T2_ref, b2_ref, o_ref):
    B = x_ref.shape[0]
    x = x_ref[...]
    T1 = T1_ref[...]
    T2 = T2_ref[...]
    acc = None
    for c in range(3):
        for d in range(5):
            A = x[:, c, d:d + 28, :].reshape(B * 28, 32)
            t = jnp.dot(A, T1[c * 5 + d], preferred_element_type=_F32)
            acc = t if acc is None else acc + t
    acc = jnp.maximum(acc + b1_ref[...], 0.0)
    pj = jnp.maximum(acc[:, :84], acc[:, 84:])
    pj = jnp.max(pj.reshape(B * 14, 2, 84), axis=1)
    p1 = pj.reshape(B, 14, 84)
    acc2 = None
    for c in range(6):
        for d in range(5):
            A2 = p1[:, d:d + 10, c * 14:(c + 1) * 14].reshape(B * 10, 14)
            t = jnp.dot(A2, T2[c * 5 + d], preferred_element_type=_F32)
            acc2 = t if acc2 is None else acc2 + t
    acc2 = jnp.maximum(acc2 + b2_ref[...], 0.0)
    qj = jnp.maximum(acc2[:, :80], acc2[:, 80:])
    qj = jnp.max(qj.reshape(B * 5, 2, 80), axis=1)
    o_ref[...] = qj.reshape(B, 5, 80)


def _combine_body(fi_ref, fr_ref, gm_ref, Wfs_ref, bf_ref, Wgs_ref, bg_ref,
                  Wmt_ref, bm_ref, o_ref):
    B = fi_ref.shape[0]
    S = fr_ref.shape[0] // B
    H = bf_ref.shape[1]
    C = bm_ref.shape[1]
    fi = fi_ref[...]
    fr = fr_ref[...]
    Wfs = Wfs_ref[...]
    Wgs = Wgs_ref[...]
    xs = rs = gs = None
    for i in range(5):
        a = jnp.dot(fi[:, i, :], Wfs[i], preferred_element_type=_F32)
        r = jnp.dot(fr[:, i, :], Wfs[i], preferred_element_type=_F32)
        g = jnp.dot(fi[:, i, :], Wgs[i], preferred_element_type=_F32)
        xs = a if xs is None else xs + a
        rs = r if rs is None else rs + r
        gs = g if gs is None else gs + g
    x = xs + bf_ref[...]
    rx = (rs + bf_ref[...]).reshape(B, S, H)
    g = gs + bg_ref[...]
    diff = x[:, None, :] - rx
    ml = jnp.dot(gm_ref[...].reshape(B * S, C), Wmt_ref[...],
                 preferred_element_type=_F32) + bm_ref[...]
    lg = ml.reshape(B, S, C)
    for h in range(H):
        gh = g[:, h * C:(h + 1) * C][:, None, :]
        lg = lg + diff[:, :, h:h + 1] * gh
    nrm = jnp.sqrt(jnp.sum(diff * diff, axis=-1))
    mx = jnp.max(-nrm, axis=-1, keepdims=True)
    e = jnp.exp(-nrm - mx)
    dist = e / jnp.sum(e, axis=-1, keepdims=True)
    o_ref[...] = jnp.sum(lg * dist[:, :, None], axis=1)


def kernel(img, train, mem, w1, b1, w2, b2, Wf, bf, Wg, bg, Wm, bm, idx, refs):
    n, S = refs.shape
    H = bf.shape[0]
    C = bm.shape[0]
    nr = n * S

    s1 = np.zeros((5, 32, 28), np.float32)
    s2 = np.zeros((5, 14, 10), np.float32)
    for dj in range(5):
        s1[dj, dj + np.arange(28), np.arange(28)] = 1.0
        s2[dj, dj + np.arange(10), np.arange(10)] = 1.0
    T1 = jnp.einsum('Dwj,ocdD->cdwoj', jnp.asarray(s1), w1).reshape(15, 32, 168)
    T2 = jnp.einsum('Dwj,ocdD->cdwoj', jnp.asarray(s2), w2).reshape(30, 14, 160)
    O1, J1 = np.meshgrid(np.arange(6), np.arange(28), indexing='ij')
    p1m = np.empty(168, np.int32)
    p1m[((J1 % 2) * 84 + O1 * 14 + J1 // 2).ravel()] = np.arange(168)
    O2, J2 = np.meshgrid(np.arange(16), np.arange(10), indexing='ij')
    p2m = np.empty(160, np.int32)
    p2m[((J2 % 2) * 80 + O2 * 5 + J2 // 2).ravel()] = np.arange(160)
    T1 = T1[:, :, p1m]
    T2 = T2[:, :, p2m]
    b1r = jnp.tile(jnp.repeat(b1, 14), 2).reshape(1, 168)
    b2r = jnp.tile(jnp.repeat(b2, 5), 2).reshape(1, 160)
    perm = np.arange(400).reshape(16, 5, 5).transpose(1, 0, 2).reshape(400)
    Wfs = Wf[:, perm].reshape(H, 5, 80).transpose(1, 2, 0)
    Wgs = Wg[:, perm].reshape(H * C, 5, 80).transpose(1, 2, 0)
    bf2 = bf.reshape(1, H)
    bg2 = bg.reshape(1, H * C)
    Wmt = Wm.T
    bm2 = bm.reshape(1, C)
    refs_flat = refs.reshape(-1).astype(jnp.int32)
    mem3 = mem.reshape(mem.shape[0], 1, C)

    img_specs = [
        pl.BlockSpec((1, 3, 32, 32), lambda i, r, g=g: (r[i * _G + g], 0, 0, 0))
        for g in range(_G)
    ]
    mem_specs = [
        pl.BlockSpec((1, 1, C), lambda i, r, g=g: (r[i * _G + g], 0, 0))
        for g in range(_G)
    ]
    gi, gm = pl.pallas_call(
        _gather_body,
        grid_spec=pltpu.PrefetchScalarGridSpec(
            num_scalar_prefetch=1,
            grid=(nr // _G,),
            in_specs=img_specs + mem_specs,
            out_specs=[
                pl.BlockSpec((_G, 3, 32, 32), lambda i, r: (i, 0, 0, 0)),
                pl.BlockSpec((_G, 1, C), lambda i, r: (i, 0, 0)),
            ],
        ),
        out_shape=[
            jax.ShapeDtypeStruct((nr, 3, 32, 32), _F32),
            jax.ShapeDtypeStruct((nr, 1, C), _F32),
        ],
        compiler_params=pltpu.CompilerParams(
            dimension_semantics=("parallel",)),
        name="mad_gather",
    )(refs_flat, *([train] * _G), *([mem3] * _G))

    def featurize(x, B):
        nb = x.shape[0] // B
        return pl.pallas_call(
            _featurize_body,
            grid=(nb,),
            in_specs=[
                pl.BlockSpec((B, 3, 32, 32), lambda b: (b, 0, 0, 0)),
                pl.BlockSpec((15, 32, 168), lambda b: (0, 0, 0)),
                pl.BlockSpec((1, 168), lambda b: (0, 0)),
                pl.BlockSpec((30, 14, 160), lambda b: (0, 0, 0)),
                pl.BlockSpec((1, 160), lambda b: (0, 0)),
            ],
            out_specs=pl.BlockSpec((B, 5, 80), lambda b: (b, 0, 0)),
            out_shape=jax.ShapeDtypeStruct((x.shape[0], 5, 80), _F32),
            compiler_params=pltpu.CompilerParams(
                dimension_semantics=("parallel",)),
            name="mad_featurize",
        )(x, T1, b1r, T2, b2r)

    fi = featurize(img, 128)
    fr = featurize(gi, 128)

    BN = 128
    out = pl.pallas_call(
        _combine_body,
        grid=(n // BN,),
        in_specs=[
            pl.BlockSpec((BN, 5, 80), lambda b: (b, 0, 0)),
            pl.BlockSpec((BN * S, 5, 80), lambda b: (b, 0, 0)),
            pl.BlockSpec((BN * S, 1, C), lambda b: (b, 0, 0)),
            pl.BlockSpec((5, 80, H), lambda b: (0, 0, 0)),
            pl.BlockSpec((1, H), lambda b: (0, 0)),
            pl.BlockSpec((5, 80, H * C), lambda b: (0, 0, 0)),
            pl.BlockSpec((1, H * C), lambda b: (0, 0)),
            pl.BlockSpec((C, C), lambda b: (0, 0)),
            pl.BlockSpec((1, C), lambda b: (0, 0)),
        ],
        out_specs=pl.BlockSpec((BN, C), lambda b: (b, 0)),
        out_shape=jax.ShapeDtypeStruct((n, C), _F32),
        compiler_params=pltpu.CompilerParams(
            dimension_semantics=("parallel",)),
        name="mad_combine",
    )(fi, fr, gm, Wfs, bf2, Wgs, bg2, Wmt, bm2)
    return out
